# Initial kernel scaffold; baseline (speedup 1.0000x reference)
import jax
import jax.numpy as jnp
from jax.experimental import pallas as pl


def kernel(view0, view1, proj_W, proj_b, router_W, expert_keys, W1, b1, W2, b2):
    raise NotImplementedError("write your pallas kernel here")


# speedup vs baseline: 2.9473x; 2.9473x over previous
import jax
import jax.numpy as jnp
from jax.experimental import pallas as pl
from jax.experimental.pallas import tpu as pltpu

DM = 768
DF = 3072
NE = 8
NT = 2048
NV = 2
BM = 256
NB = NT // BM


def _routing_body(v_ref, pw_ref, pb_ref, rw_ref, keys_ref, h_ref, g_ref):
    v = v_ref[0]
    h = jnp.dot(v, pw_ref[0], preferred_element_type=jnp.float32) + pb_ref[0]
    r = jnp.dot(h, rw_ref[0], preferred_element_type=jnp.float32)
    keys = keys_ref[...]
    diff = r[:, None, :] - keys[None, :, :]
    d2 = jnp.sum(diff * diff, axis=-1)
    logits = -jnp.sqrt(d2 + 1e-12)
    iota = jax.lax.broadcasted_iota(jnp.int32, (BM, NE), 1)
    m1 = jnp.max(logits, axis=1, keepdims=True)
    i1 = jnp.min(jnp.where(logits == m1, iota, NE), axis=1, keepdims=True)
    l2 = jnp.where(iota == i1, -1e30, logits)
    m2 = jnp.max(l2, axis=1, keepdims=True)
    i2 = jnp.min(jnp.where(l2 == m2, iota, NE), axis=1, keepdims=True)
    e2 = jnp.exp(m2 - m1)
    den = 1.0 + e2
    g = jnp.where(iota == i1, 1.0 / den, 0.0) + jnp.where(iota == i2, e2 / den, 0.0)
    h_ref[...] = h
    g_ref[...] = g


def _ffn_body(h0_ref, h1_ref, g0_ref, g1_ref, w1_ref, b1_ref, w2_ref, b2_ref,
              out_ref, acc_ref):
    e = pl.program_id(0)
    t = pl.program_id(1)
    iota = jax.lax.broadcasted_iota(jnp.int32, (BM, NE), 1)

    def one(h_ref, g_ref):
        x = h_ref[...]
        mid = jnp.dot(x, w1_ref[0], preferred_element_type=jnp.float32) + b1_ref[0]
        mid = 0.5 * mid * (1.0 + jax.lax.erf(mid * 0.7071067811865476))
        y = jnp.dot(mid, w2_ref[0], preferred_element_type=jnp.float32) + b2_ref[0]
        ge = jnp.sum(g_ref[...] * (iota == e).astype(jnp.float32), axis=1,
                     keepdims=True)
        return ge * y

    upd = one(h0_ref, g0_ref) + one(h1_ref, g1_ref)
    sl = pl.ds(t * BM, BM)

    @pl.when(e == 0)
    def _():
        acc_ref[sl, :] = upd

    @pl.when(e != 0)
    def _():
        acc_ref[sl, :] = acc_ref[sl, :] + upd

    out_ref[...] = acc_ref[sl, :]


def _routing(V, proj_W, proj_b, router_W, expert_keys):
    return pl.pallas_call(
        _routing_body,
        grid=(NV, NB),
        in_specs=[
            pl.BlockSpec((1, BM, DM), lambda v, t: (v, t, 0)),
            pl.BlockSpec((1, DM, DM), lambda v, t: (v, 0, 0)),
            pl.BlockSpec((1, 1, DM), lambda v, t: (v, 0, 0)),
            pl.BlockSpec((1, DM, NE), lambda v, t: (v, 0, 0)),
            pl.BlockSpec((NE, NE), lambda v, t: (0, 0)),
        ],
        out_specs=[
            pl.BlockSpec((BM, DM), lambda v, t: (v * NB + t, 0)),
            pl.BlockSpec((BM, NE), lambda v, t: (v * NB + t, 0)),
        ],
        out_shape=[
            jax.ShapeDtypeStruct((NV * NT, DM), jnp.float32),
            jax.ShapeDtypeStruct((NV * NT, NE), jnp.float32),
        ],
    )(V, proj_W, proj_b.reshape(NV, 1, DM), router_W, expert_keys)


def _ffn(H, G, W1, b1, W2, b2):
    return pl.pallas_call(
        _ffn_body,
        grid=(NE, NB),
        in_specs=[
            pl.BlockSpec((BM, DM), lambda e, t: (t, 0)),
            pl.BlockSpec((BM, DM), lambda e, t: (NB + t, 0)),
            pl.BlockSpec((BM, NE), lambda e, t: (t, 0)),
            pl.BlockSpec((BM, NE), lambda e, t: (NB + t, 0)),
            pl.BlockSpec((1, DM, DF), lambda e, t: (e, 0, 0)),
            pl.BlockSpec((1, 1, DF), lambda e, t: (e, 0, 0)),
            pl.BlockSpec((1, DF, DM), lambda e, t: (e, 0, 0)),
            pl.BlockSpec((1, 1, DM), lambda e, t: (e, 0, 0)),
        ],
        out_specs=pl.BlockSpec((BM, DM), lambda e, t: (t, 0)),
        out_shape=jax.ShapeDtypeStruct((NT, DM), jnp.float32),
        scratch_shapes=[pltpu.VMEM((NT, DM), jnp.float32)],
    )(H, H, G, G, W1, b1.reshape(NE, 1, DF), W2, b2.reshape(NE, 1, DM))


def kernel(view0, view1, proj_W, proj_b, router_W, expert_keys, W1, b1, W2, b2):
    V = jnp.concatenate([view0, view1], axis=0)
    H, G = _routing(V, proj_W, proj_b, router_W, expert_keys)
    out = _ffn(H, G, W1, b1, W2, b2)
    return out.reshape(1, NT, DM)
